# baseline (speedup 1.0000x reference)
import functools

import numpy as np
import jax
import jax.numpy as jnp
from jax import lax
from jax.experimental import pallas as pl
from jax.experimental.pallas import tpu as pltpu
from jax.experimental.pallas import tpu_sc as plsc

_B, _S, _D = 4, 2048, 1024
_H = _D // 2
_SCALE = float(np.sqrt(_D))
_NW = 32
_SPW = _S // _NW
_C = 8
_NCH = _SPW // _C
_RC = _B * _C
_L = 16
_NBUF = 3


def _pos_seed_rates() -> np.ndarray:
    depth = _D / 2
    rates = (1 / 10000 ** (np.arange(depth)[np.newaxis, :] / depth))[0]
    seeds_s = np.arange(_NW)[:, np.newaxis] * _SPW * rates[np.newaxis, :]
    seed_sin = np.sin(seeds_s)
    seed_cos = np.cos(seeds_s)
    step_cos = np.cos(rates)[np.newaxis, :].repeat(_NW, axis=0)
    step_sin = np.sin(rates)[np.newaxis, :].repeat(_NW, axis=0)

    def blockify(a, b):
        ab = np.stack([a.reshape(_NW, _H // _L, _L),
                       b.reshape(_NW, _H // _L, _L)], axis=2)
        return ab.reshape(_NW, _D)

    seed = blockify(seed_sin, seed_cos)
    step = blockify(step_cos, step_sin)
    return np.concatenate([seed, step], axis=1).astype(np.float32)


_POSROT = _pos_seed_rates()

_mesh = plsc.VectorSubcoreMesh(core_axis_name="c", subcore_axis_name="s")


@functools.partial(
    pl.kernel,
    mesh=_mesh,
    out_type=jax.ShapeDtypeStruct((_B, _S, _D), jnp.float32),
    scratch_types=(
        [pltpu.VMEM((_B, _SPW), jnp.int32)]
        + [pltpu.VMEM((_RC, _D), jnp.float32) for _ in range(_NBUF)]
        + [pltpu.VMEM((2 * _D,), jnp.float32)]
        + [pltpu.SemaphoreType.DMA for _ in range(2 * _NBUF + 1)]
    ),
)
def _emb_kernel(x_hbm, table_hbm, posrot_hbm, out_hbm,
                idx_v, buf0, buf1, buf2, pr_v,
                g0, g1, g2, s0, s1, s2, p0):
    bufs = (buf0, buf1, buf2)
    gsems = (g0, g1, g2)
    ssems = (s0, s1, s2)

    wid = lax.axis_index("s") * 2 + lax.axis_index("c")
    sbase = wid * _SPW
    idx_hs = [
        pltpu.async_copy(x_hbm.at[bb, pl.ds(sbase, _SPW)], idx_v.at[bb], p0)
        for bb in range(_B)
    ] + [pltpu.async_copy(posrot_hbm.at[wid], pr_v, p0)]

    def start_chunk(g):
        buf = bufs[g % _NBUF]
        return [
            pltpu.async_copy(
                table_hbm.at[idx_v.at[bb, pl.ds(g * _C, _C)]],
                buf.at[pl.ds(bb * _C, _C)],
                gsems[g % _NBUF])
            for bb in range(_B)
        ]

    for h in idx_hs:
        h.wait()

    pending = {}
    pending_stores = {b: [] for b in range(_NBUF)}
    pending[0] = start_chunk(0)

    for g in range(_NCH):
        b = g % _NBUF
        if g + 1 < _NCH:
            nb = (g + 1) % _NBUF
            for h in pending_stores[nb]:
                h.wait()
            pending_stores[nb] = []
            pending[g + 1] = start_chunk(g + 1)
        for h in pending.pop(g):
            h.wait()

        buf = bufs[b]

        def row_body(r, _, buf=buf):
            @plsc.parallel_loop(0, _H // _L, unroll=4)
            def _(j):
                off = j * 2 * _L
                ps = pr_v[pl.ds(off, _L)]
                pc = pr_v[pl.ds(off + _L, _L)]
                sl0 = pl.ds(j * _L, _L)
                sl1 = pl.ds(_H + j * _L, _L)
                for bb in range(_B):
                    row = bb * _C + r
                    buf[row, sl0] = buf[row, sl0] * _SCALE + ps
                    buf[row, sl1] = buf[row, sl1] * _SCALE + pc
                rc = pr_v[pl.ds(_D + off, _L)]
                rs = pr_v[pl.ds(_D + off + _L, _L)]
                pr_v[pl.ds(off, _L)] = ps * rc + pc * rs
                pr_v[pl.ds(off + _L, _L)] = pc * rc - ps * rs
            return 0

        lax.fori_loop(0, _C, row_body, 0)

        for bb in range(_B):
            h = pltpu.async_copy(
                buf.at[pl.ds(bb * _C, _C)],
                out_hbm.at[bb, pl.ds(sbase + g * _C, _C)],
                ssems[b])
            pending_stores[b].append(h)

    for b in range(_NBUF):
        for h in pending_stores[b]:
            h.wait()


def kernel(x, table):
    return _emb_kernel(x, table, jnp.asarray(_POSROT))


# speedup vs baseline: 1.1213x; 1.1213x over previous
import functools

import numpy as np
import jax
import jax.numpy as jnp
from jax import lax
from jax.experimental import pallas as pl
from jax.experimental.pallas import tpu as pltpu
from jax.experimental.pallas import tpu_sc as plsc

_B, _S, _D = 4, 2048, 1024
_H = _D // 2
_SCALE = float(np.sqrt(_D))
_NW = 32
_SPW = _S // _NW
_C = 8
_NCH = _SPW // _C
_RC = _B * _C
_L = 16
_NBUF = 3


def _pos_seed_rates() -> np.ndarray:
    depth = _D / 2
    rates = (1 / 10000 ** (np.arange(depth)[np.newaxis, :] / depth))[0]
    seeds_s = np.arange(_NW)[:, np.newaxis] * _SPW * rates[np.newaxis, :]
    seed_sin = np.sin(seeds_s)
    seed_cos = np.cos(seeds_s)
    step_cos = np.cos(rates)[np.newaxis, :].repeat(_NW, axis=0)
    step_sin = np.sin(rates)[np.newaxis, :].repeat(_NW, axis=0)

    def blockify(a, b):
        ab = np.stack([a.reshape(_NW, _H // _L, _L),
                       b.reshape(_NW, _H // _L, _L)], axis=2)
        return ab.reshape(_NW, _D)

    seed = blockify(seed_sin, seed_cos)
    step = blockify(step_cos, step_sin)
    return np.concatenate([seed, step], axis=1).astype(np.float32)


_POSROT = _pos_seed_rates()

_mesh = plsc.VectorSubcoreMesh(core_axis_name="c", subcore_axis_name="s")


@functools.partial(
    pl.kernel,
    mesh=_mesh,
    out_type=jax.ShapeDtypeStruct((_B, _S, _D), jnp.float32),
    scratch_types=(
        [pltpu.VMEM((_B, _SPW), jnp.int32)]
        + [pltpu.VMEM((_RC, _D), jnp.float32) for _ in range(_NBUF)]
        + [pltpu.VMEM((2 * _D,), jnp.float32)]
        + [pltpu.SemaphoreType.DMA for _ in range(2 * _NBUF + 1)]
    ),
)
def _emb_kernel(x_hbm, table_hbm, posrot_hbm, out_hbm,
                idx_v, buf0, buf1, buf2, pr_v,
                g0, g1, g2, s0, s1, s2, p0):
    bufs = (buf0, buf1, buf2)
    gsems = (g0, g1, g2)
    ssems = (s0, s1, s2)

    wid = lax.axis_index("s") * 2 + lax.axis_index("c")
    sbase = wid * _SPW
    idx_hs = [
        pltpu.async_copy(x_hbm.at[bb, pl.ds(sbase, _SPW)], idx_v.at[bb], p0)
        for bb in range(_B)
    ] + [pltpu.async_copy(posrot_hbm.at[wid], pr_v, p0)]

    def start_chunk(g):
        buf = bufs[g % _NBUF]
        return [
            pltpu.async_copy(
                table_hbm.at[idx_v.at[bb, pl.ds(g * _C, _C)]],
                buf.at[pl.ds(bb * _C, _C)],
                gsems[g % _NBUF])
            for bb in range(_B)
        ]

    for h in idx_hs:
        h.wait()

    pending = {}
    pending_stores = {b: [] for b in range(_NBUF)}
    pending[0] = start_chunk(0)

    for g in range(_NCH):
        b = g % _NBUF
        if g + 1 < _NCH:
            nb = (g + 1) % _NBUF
            for h in pending_stores[nb]:
                h.wait()
            pending_stores[nb] = []
            pending[g + 1] = start_chunk(g + 1)
        for h in pending.pop(g):
            h.wait()

        buf = bufs[b]

        def row_body(r, _, buf=buf):
            @plsc.parallel_loop(0, _H // _L, unroll=2)
            def _(j):
                off = j * 2 * _L
                ps = pr_v[pl.ds(off, _L)]
                pc = pr_v[pl.ds(off + _L, _L)]
                sl0 = pl.ds(j * _L, _L)
                sl1 = pl.ds(_H + j * _L, _L)
                for bb in range(_B):
                    row = bb * _C + r
                    buf[row, sl0] = buf[row, sl0] * _SCALE + ps
                    buf[row, sl1] = buf[row, sl1] * _SCALE + pc
                rc = pr_v[pl.ds(_D + off, _L)]
                rs = pr_v[pl.ds(_D + off + _L, _L)]
                pr_v[pl.ds(off, _L)] = ps * rc + pc * rs
                pr_v[pl.ds(off + _L, _L)] = pc * rc - ps * rs
            return 0

        lax.fori_loop(0, _C, row_body, 0)

        for bb in range(_B):
            h = pltpu.async_copy(
                buf.at[pl.ds(bb * _C, _C)],
                out_hbm.at[bb, pl.ds(sbase + g * _C, _C)],
                ssems[b])
            pending_stores[b].append(h)

    for b in range(_NBUF):
        for h in pending_stores[b]:
            h.wait()


def kernel(x, table):
    return _emb_kernel(x, table, jnp.asarray(_POSROT))


# speedup vs baseline: 1.1323x; 1.0098x over previous
import functools

import numpy as np
import jax
import jax.numpy as jnp
from jax import lax
from jax.experimental import pallas as pl
from jax.experimental.pallas import tpu as pltpu
from jax.experimental.pallas import tpu_sc as plsc

_B, _S, _D = 4, 2048, 1024
_H = _D // 2
_SCALE = float(np.sqrt(_D))
_NW = 32
_SPW = _S // _NW
_C = 8
_NCH = _SPW // _C
_RC = _B * _C
_L = 16
_NBUF = 3


def _pos_seed_rates() -> np.ndarray:
    depth = _D / 2
    rates = (1 / 10000 ** (np.arange(depth)[np.newaxis, :] / depth))[0]
    seeds_s = np.arange(_NW)[:, np.newaxis] * _SPW * rates[np.newaxis, :]
    seed_sin = np.sin(seeds_s)
    seed_cos = np.cos(seeds_s)
    step_cos = np.cos(rates)[np.newaxis, :].repeat(_NW, axis=0)
    step_sin = np.sin(rates)[np.newaxis, :].repeat(_NW, axis=0)

    def blockify(a, b):
        ab = np.stack([a.reshape(_NW, _H // _L, _L),
                       b.reshape(_NW, _H // _L, _L)], axis=2)
        return ab.reshape(_NW, _D)

    seed = blockify(seed_sin, seed_cos)
    step = blockify(step_cos, step_sin)
    return np.concatenate([seed, step], axis=1).astype(np.float32)


_POSROT = _pos_seed_rates()

_mesh = plsc.VectorSubcoreMesh(core_axis_name="c", subcore_axis_name="s")


@functools.partial(
    pl.kernel,
    mesh=_mesh,
    out_type=jax.ShapeDtypeStruct((_B, _S, _D), jnp.float32),
    scratch_types=(
        [pltpu.VMEM((_B, _SPW), jnp.int32)]
        + [pltpu.VMEM((_RC, _D), jnp.float32) for _ in range(_NBUF)]
        + [pltpu.VMEM((2 * _D,), jnp.float32)]
        + [pltpu.SemaphoreType.DMA for _ in range(2 * _NBUF + 1)]
    ),
)
def _emb_kernel(x_hbm, table_hbm, posrot_hbm, out_hbm,
                idx_v, buf0, buf1, buf2, pr_v,
                g0, g1, g2, s0, s1, s2, p0):
    bufs = (buf0, buf1, buf2)
    gsems = (g0, g1, g2)
    ssems = (s0, s1, s2)

    wid = lax.axis_index("s") * 2 + lax.axis_index("c")
    sbase = wid * _SPW
    idx_hs = [
        pltpu.async_copy(x_hbm.at[bb, pl.ds(sbase, _SPW)], idx_v.at[bb], p0)
        for bb in range(_B)
    ] + [pltpu.async_copy(posrot_hbm.at[wid], pr_v, p0)]

    def start_chunk(g):
        buf = bufs[g % _NBUF]
        return [
            pltpu.async_copy(
                table_hbm.at[idx_v.at[bb, pl.ds(g * _C, _C)]],
                buf.at[pl.ds(bb * _C, _C)],
                gsems[g % _NBUF])
            for bb in range(_B)
        ]

    for h in idx_hs:
        h.wait()

    pending = {}
    pending_stores = {b: [] for b in range(_NBUF)}
    pending[0] = start_chunk(0)

    for g in range(_NCH):
        b = g % _NBUF
        if g + 1 < _NCH:
            nb = (g + 1) % _NBUF
            for h in pending_stores[nb]:
                h.wait()
            pending_stores[nb] = []
            pending[g + 1] = start_chunk(g + 1)
        for h in pending.pop(g):
            h.wait()

        buf = bufs[b]

        def row_body(r, _, buf=buf):
            @plsc.parallel_loop(0, _H // _L, unroll=1)
            def _(j):
                off = j * 2 * _L
                ps = pr_v[pl.ds(off, _L)]
                pc = pr_v[pl.ds(off + _L, _L)]
                sl0 = pl.ds(j * _L, _L)
                sl1 = pl.ds(_H + j * _L, _L)
                for bb in range(_B):
                    row = bb * _C + r
                    buf[row, sl0] = buf[row, sl0] * _SCALE + ps
                    buf[row, sl1] = buf[row, sl1] * _SCALE + pc
                rc = pr_v[pl.ds(_D + off, _L)]
                rs = pr_v[pl.ds(_D + off + _L, _L)]
                pr_v[pl.ds(off, _L)] = ps * rc + pc * rs
                pr_v[pl.ds(off + _L, _L)] = pc * rc - ps * rs
            return 0

        lax.fori_loop(0, _C, row_body, 0)

        for bb in range(_B):
            h = pltpu.async_copy(
                buf.at[pl.ds(bb * _C, _C)],
                out_hbm.at[bb, pl.ds(sbase + g * _C, _C)],
                ssems[b])
            pending_stores[b].append(h)

    for b in range(_NBUF):
        for h in pending_stores[b]:
            h.wait()


def kernel(x, table):
    return _emb_kernel(x, table, jnp.asarray(_POSROT))
